# baseline (speedup 1.0000x reference)
import functools

import jax
import jax.numpy as jnp
from jax import lax
from jax.experimental import pallas as pl
from jax.experimental.pallas import tpu as pltpu
from jax.experimental.pallas import tpu_sc as plsc

N = 10000
E = 160000
C = 256
DM = 64
OMEGA = 0.1
K_HYP = 1.0
B_IDX = 2048
HN = 10240
BLK = 1000
GRID = N // BLK

_mesh = plsc.VectorSubcoreMesh(core_axis_name="c", subcore_axis_name="s")


@functools.partial(
    pl.kernel,
    out_type=[jax.ShapeDtypeStruct((2, 16, 1, HN), jnp.float32),
              jax.ShapeDtypeStruct((2, 16, 1, HN), jnp.float32)],
    mesh=_mesh,
    scratch_types=[
        pltpu.VMEM((1, 5120), jnp.int32),
        pltpu.VMEM((1, 64), jnp.int32),
        pltpu.VMEM((HN,), jnp.float32),
        pltpu.VMEM((HN,), jnp.float32),
    ],
    compiler_params=pltpu.CompilerParams(needs_layout_passes=False),
)
def _sc_hist(dstp_hbm, idxp_hbm, outd_hbm, outi_hbm, dst_v, idx_v,
             hd_v, hi_v):
    c = lax.axis_index("c")
    s = lax.axis_index("s")

    def zero(j, carry):
        hd_v[pl.ds(16 * j, 16)] = jnp.zeros((16,), jnp.float32)
        hi_v[pl.ds(16 * j, 16)] = jnp.zeros((16,), jnp.float32)
        return carry

    lax.fori_loop(0, HN // 16, zero, 0)
    pltpu.sync_copy(dstp_hbm.at[c, s], dst_v)
    pltpu.sync_copy(idxp_hbm.at[c, s], idx_v)

    ones = jnp.full((16,), 1.0, jnp.float32)

    def accum_d(j, carry):
        v = dst_v[0, pl.ds(16 * j, 16)]
        plsc.addupdate_scatter(hd_v, [v], ones)
        return carry

    lax.fori_loop(0, 5120 // 16, accum_d, 0)

    def accum_i(j, carry):
        v = idx_v[0, pl.ds(16 * j, 16)]
        plsc.addupdate_scatter(hi_v, [v], ones)
        return carry

    lax.fori_loop(0, 64 // 16, accum_i, 0)
    pltpu.sync_copy(hd_v, outd_hbm.at[c, s, 0])
    pltpu.sync_copy(hi_v, outi_hbm.at[c, s, 0])


@functools.partial(
    pl.kernel,
    out_type=jax.ShapeDtypeStruct((2, 2, N, 128), jnp.float32),
    mesh=_mesh,
    scratch_types=[
        pltpu.VMEM((128, 80), jnp.int32),
        pltpu.VMEM((128, 80), jnp.int32),
        pltpu.VMEM((80, 128), jnp.float32),
        pltpu.VMEM_SHARED((N, 128), jnp.float32),
        pltpu.SemaphoreType.DMA,
        pltpu.SemaphoreType.DMA,
        pltpu.SemaphoreType.DMA,
        pltpu.SemaphoreType.DMA,
        pltpu.SemaphoreType.DMA,
    ],
)
def _sc_scatter(t_hbm, edg_hbm, z_hbm, out_hbm,
                gi_v, si_v, rbuf, acc, g0, g1, s0, s1, wsem):
    gsems = [g0, g1]
    ssems = [s0, s1]
    QS = [(0, 40), (40, 40)]
    c = lax.axis_index("c")
    s = lax.axis_index("s")
    pltpu.sync_copy(edg_hbm.at[c, s], gi_v)
    pltpu.sync_copy(edg_hbm.at[1 - c, s], si_v)
    rbase = s * 640
    nch = jnp.where(s < 15, 8, 5)
    for p in range(2):

        def zch(j, carry):
            pltpu.async_copy(z_hbm, acc.at[pl.ds(rbase + 80 * j, 80)], wsem)
            return carry

        def wdrain(j, carry):
            pltpu.make_async_copy(z_hbm, acc.at[pl.ds(rbase, 80)],
                                  wsem).wait()
            return carry

        lax.fori_loop(0, nch, zch, 0)
        lax.fori_loop(0, nch, wdrain, 0)
        plsc.subcore_barrier()

        def gat(j, m):
            return pltpu.async_copy(
                t_hbm.at[c, p].at[gi_v.at[4 * j + m // 2,
                                          pl.ds((m % 2) * 40, 40)]],
                rbuf.at[pl.ds((m % 2) * 40, 40)], gsems[m % 2])

        def sct(j, m):
            return pltpu.async_copy(
                rbuf.at[pl.ds((m % 2) * 40, 40)],
                acc.at[si_v.at[4 * j + m // 2, pl.ds((m % 2) * 40, 40)]],
                ssems[m % 2], add=True)

        def chunk8(j, carry):
            g = {0: gat(j, 0), 1: gat(j, 1)}
            for m in range(8):
                g[m].wait()
                s = sct(j, m)
                s.wait()
                if m + 2 <= 7:
                    g[m + 2] = gat(j, m + 2)
            return carry

        lax.fori_loop(0, 31, chunk8, 0)
        gt0 = gat(31, 0)
        gt1 = gat(31, 1)
        gt0.wait()
        st0 = sct(31, 0)
        gt1.wait()
        st0.wait()
        st1 = sct(31, 1)
        st1.wait()
        plsc.subcore_barrier()

        def wch(j, carry):
            pltpu.async_copy(acc.at[pl.ds(rbase + 80 * j, 80)],
                             out_hbm.at[c, p, pl.ds(rbase + 80 * j, 80)],
                             wsem)
            return carry

        def wdrain2(j, carry):
            pltpu.make_async_copy(z_hbm, acc.at[pl.ds(rbase, 80)],
                                  wsem).wait()
            return carry

        lax.fori_loop(0, nch, wch, 0)
        lax.fori_loop(0, nch, wdrain2, 0)
        plsc.subcore_barrier()


def _tc_dense_body(x_ref, wc_ref, deg_ref, hd_ref, t_ref, dis_ref,
                   dsum_ref):
    i = pl.program_id(0)
    h = jnp.dot(x_ref[...], wc_ref[...], preferred_element_type=jnp.float32)
    degsl = jnp.sum(hd_ref[...], axis=1) + 1.0
    dis = lax.rsqrt(degsl)
    hs = h * dis[:, None]
    for p in range(2):
        t_ref[0, p] = hs[:, 128 * p:128 * (p + 1)]
        t_ref[1, p] = h[:, 128 * p:128 * (p + 1)]
    dis_ref[...] = dis[:, None]
    d_i = deg_ref[...]

    @pl.when(i == 0)
    def _():
        dsum_ref[...] = jnp.zeros_like(dsum_ref)

    dsum_ref[...] += jnp.sum(d_i.astype(jnp.float32))


def _tc_dense(x, W_conv, degree, hd):
    return pl.pallas_call(
        _tc_dense_body,
        grid=(GRID,),
        in_specs=[
            pl.BlockSpec((BLK, C), lambda i: (i, 0)),
            pl.BlockSpec((C, C), lambda i: (0, 0)),
            pl.BlockSpec((BLK, 1), lambda i: (i, 0)),
            pl.BlockSpec((BLK, 32), lambda i: (i, 0)),
        ],
        out_specs=[
            pl.BlockSpec((2, 2, BLK, 128), lambda i: (0, 0, i, 0)),
            pl.BlockSpec((BLK, 1), lambda i: (i, 0)),
            pl.BlockSpec((1, 1), lambda i: (0, 0)),
        ],
        out_shape=[
            jax.ShapeDtypeStruct((2, 2, N, 128), jnp.float32),
            jax.ShapeDtypeStruct((N, 1), jnp.float32),
            jax.ShapeDtypeStruct((1, 1), jnp.float32),
        ],
    )(x, W_conv, degree, hd)


def _tc_final_body(ac_ref, t_ref, dis_ref, deg_ref, pe_ref, wg_ref,
                   wb_ref, bg_ref, bb_ref, wa_ref, wr_ref, hi_ref, dsum_ref,
                   out_ref, lb_ref, lf_ref):
    i = pl.program_id(0)
    agg = jnp.concatenate([ac_ref[1, 0], ac_ref[1, 1]], axis=1) * (DM ** 0.5)
    deg_f = deg_ref[...].astype(jnp.float32)
    safe = jnp.where(deg_f == 0, 1.0, deg_f)
    i_n = jnp.where(deg_f == 0, 0.0, agg / safe)
    A = jnp.dot(i_n, wa_ref[...], preferred_element_type=jnp.float32)
    Rv = jnp.dot(i_n, wr_ref[...], preferred_element_type=jnp.float32)
    iota = lax.broadcasted_iota(jnp.int32, (BLK, 128), 1)
    onehot = (deg_ref[...] == iota).astype(jnp.float32)
    m_dv = jnp.dot(onehot, pe_ref[...], preferred_element_type=jnp.float32)

    def lrelu(v):
        return jnp.where(v >= 0, v, 0.01 * v)

    gam = lrelu(
        jnp.dot(m_dv, wg_ref[...], preferred_element_type=jnp.float32)
        + bg_ref[...])
    bet = lrelu(
        jnp.dot(m_dv, wb_ref[...], preferred_element_type=jnp.float32)
        + bb_ref[...])
    gp1 = gam + 1.0
    b_add = gp1 * A + bet
    b_rev = gp1 * Rv + bet
    Kv = dsum_ref[0, 0] * (K_HYP / N)
    R = (deg_f < Kv).astype(jnp.float32)
    bias = OMEGA * (R * b_add - (1.0 - R) * b_rev)
    hfull = jnp.concatenate([t_ref[0, 0], t_ref[0, 1]], axis=1)
    conv1 = jnp.concatenate([ac_ref[0, 0], ac_ref[0, 1]], axis=1)
    dis = dis_ref[...]
    out_ref[...] = conv1 * dis + hfull * (dis * dis) + bias
    na = jnp.sqrt(jnp.sum(b_add * b_add, axis=1, keepdims=True)) * R
    nr = jnp.sqrt(jnp.sum(b_rev * b_rev, axis=1, keepdims=True)) * (1.0 - R)
    ng = jnp.sqrt(jnp.sum(gam * gam, axis=1))
    nbv = jnp.sqrt(jnp.sum(bet * bet, axis=1))
    cnt = jnp.sum(hi_ref[...], axis=1)

    @pl.when(i == 0)
    def _():
        lb_ref[...] = jnp.zeros_like(lb_ref)
        lf_ref[...] = jnp.zeros_like(lf_ref)

    lb_ref[...] += jnp.sum(cnt * (na + nr)[:, 0])
    lf_ref[...] += jnp.sum(cnt * (ng + nbv))


def _tc_final(ac, t, dis, degree, pe, W_gamma, W_beta, b_gamma, b_beta,
              W_add, W_rev, hi, dsum):
    return pl.pallas_call(
        _tc_final_body,
        grid=(GRID,),
        in_specs=[
            pl.BlockSpec((2, 2, BLK, 128), lambda i: (0, 0, i, 0)),
            pl.BlockSpec((1, 2, BLK, 128), lambda i: (1, 0, i, 0)),
            pl.BlockSpec((BLK, 1), lambda i: (i, 0)),
            pl.BlockSpec((BLK, 1), lambda i: (i, 0)),
            pl.BlockSpec((128, DM), lambda i: (0, 0)),
            pl.BlockSpec((DM, C), lambda i: (0, 0)),
            pl.BlockSpec((DM, C), lambda i: (0, 0)),
            pl.BlockSpec((1, C), lambda i: (0, 0)),
            pl.BlockSpec((1, C), lambda i: (0, 0)),
            pl.BlockSpec((C, C), lambda i: (0, 0)),
            pl.BlockSpec((C, C), lambda i: (0, 0)),
            pl.BlockSpec((BLK, 32), lambda i: (i, 0)),
            pl.BlockSpec((1, 1), lambda i: (0, 0)),
        ],
        out_specs=[
            pl.BlockSpec((BLK, C), lambda i: (i, 0)),
            pl.BlockSpec((1, 1), lambda i: (0, 0)),
            pl.BlockSpec((1, 1), lambda i: (0, 0)),
        ],
        out_shape=[
            jax.ShapeDtypeStruct((N, C), jnp.float32),
            jax.ShapeDtypeStruct((1, 1), jnp.float32),
            jax.ShapeDtypeStruct((1, 1), jnp.float32),
        ],
    )(ac, t, dis, degree, pe, W_gamma, W_beta, b_gamma, b_beta,
      W_add, W_rev, hi, dsum)


def kernel(x, adj, degree, idx, edge, W_conv, W_gamma, W_beta, b_gamma,
           b_beta, W_add, W_rev, PE):
    src = adj[0]
    dst = adj[1]
    pad = N + (jnp.arange(3840, dtype=jnp.int32) % 16)
    dstp = jnp.concatenate([dst, pad]).reshape(2, 16, 1, 5120)
    idxp = idx.reshape(2, 16, 1, 64)
    epad = jnp.zeros((2, 16, 3, 80), jnp.int32)
    edg = jnp.concatenate([adj.reshape(2, 16, 125, 80), epad], axis=2)

    histd, histi = _sc_hist(dstp, idxp)
    hd = histd.reshape(32, HN)[:, :N].T
    hi = histi.reshape(32, HN)[:, :N].T

    T, dis, dsum = _tc_dense(x, W_conv, degree, hd)

    z = jnp.zeros((80, 128), jnp.float32)
    ac = _sc_scatter(T, edg, z)

    output, lb, lf = _tc_final(ac, T, dis, degree, PE[:128], W_gamma,
                               W_beta, b_gamma, b_beta, W_add, W_rev, hi,
                               dsum)
    nb = float(B_IDX)
    return output, lb[0, 0] / nb, lf[0, 0] / nb


# speedup vs baseline: 1.1491x; 1.1491x over previous
import functools

import jax
import jax.numpy as jnp
from jax import lax
from jax.experimental import pallas as pl
from jax.experimental.pallas import tpu as pltpu
from jax.experimental.pallas import tpu_sc as plsc

N = 10000
E = 160000
C = 256
DM = 64
OMEGA = 0.1
K_HYP = 1.0
B_IDX = 2048
HN = 10240
BLK = 1000
GRID = N // BLK

_mesh = plsc.VectorSubcoreMesh(core_axis_name="c", subcore_axis_name="s")


@functools.partial(
    pl.kernel,
    out_type=[jax.ShapeDtypeStruct((2, 16, 1, HN), jnp.float32),
              jax.ShapeDtypeStruct((2, 16, 1, HN), jnp.float32)],
    mesh=_mesh,
    scratch_types=[
        pltpu.VMEM((1, 5120), jnp.int32),
        pltpu.VMEM((1, 64), jnp.int32),
        pltpu.VMEM((HN,), jnp.float32),
        pltpu.VMEM((HN,), jnp.float32),
    ],
    compiler_params=pltpu.CompilerParams(needs_layout_passes=False),
)
def _sc_hist(dstp_hbm, idxp_hbm, outd_hbm, outi_hbm, dst_v, idx_v,
             hd_v, hi_v):
    c = lax.axis_index("c")
    s = lax.axis_index("s")

    def zero(j, carry):
        hd_v[pl.ds(16 * j, 16)] = jnp.zeros((16,), jnp.float32)
        hi_v[pl.ds(16 * j, 16)] = jnp.zeros((16,), jnp.float32)
        return carry

    lax.fori_loop(0, HN // 16, zero, 0)
    pltpu.sync_copy(dstp_hbm.at[c, s], dst_v)
    pltpu.sync_copy(idxp_hbm.at[c, s], idx_v)

    ones = jnp.full((16,), 1.0, jnp.float32)

    def accum_d(j, carry):
        v = dst_v[0, pl.ds(16 * j, 16)]
        plsc.addupdate_scatter(hd_v, [v], ones)
        return carry

    lax.fori_loop(0, 5120 // 16, accum_d, 0)

    def accum_i(j, carry):
        v = idx_v[0, pl.ds(16 * j, 16)]
        plsc.addupdate_scatter(hi_v, [v], ones)
        return carry

    lax.fori_loop(0, 64 // 16, accum_i, 0)
    pltpu.sync_copy(hd_v, outd_hbm.at[c, s, 0])
    pltpu.sync_copy(hi_v, outi_hbm.at[c, s, 0])


@functools.partial(
    pl.kernel,
    out_type=jax.ShapeDtypeStruct((2, 2, N, 128), jnp.float32),
    mesh=_mesh,
    scratch_types=[
        pltpu.VMEM((128, 80), jnp.int32),
        pltpu.VMEM((8, 80), jnp.int32),
        pltpu.VMEM((80, 128), jnp.float32),
        pltpu.VMEM((80, 128), jnp.float32),
        pltpu.VMEM_SHARED((N, 128), jnp.float32),
        pltpu.SemaphoreType.DMA,
        pltpu.SemaphoreType.DMA,
        pltpu.SemaphoreType.DMA,
        pltpu.SemaphoreType.DMA,
        pltpu.SemaphoreType.DMA,
    ],
)
def _sc_scatter(t_hbm, edg_hbm, z_hbm, out_hbm,
                gi_v, sib, rbA, rbB, acc, g0, g1, s0, s1, wsem):
    rbufs = [rbA, rbB]
    gsems = [g0, g1]
    ssems = [s0, s1]
    QS = [(0, 40), (40, 40)]
    c = lax.axis_index("c")
    s = lax.axis_index("s")
    pltpu.sync_copy(edg_hbm.at[c, s], gi_v)
    rbase = s * 640
    nch = jnp.where(s < 15, 8, 5)
    for p in range(2):

        def zch(j, carry):
            pltpu.async_copy(z_hbm, acc.at[pl.ds(rbase + 80 * j, 80)], wsem)
            return carry

        def wdrain(j, carry):
            pltpu.make_async_copy(z_hbm, acc.at[pl.ds(rbase, 80)],
                                  wsem).wait()
            return carry

        lax.fori_loop(0, nch, zch, 0)
        lax.fori_loop(0, nch, wdrain, 0)
        plsc.subcore_barrier()

        def gat(j, m):
            return pltpu.async_copy(
                t_hbm.at[c, p].at[gi_v.at[8 * j + m]],
                rbufs[m % 2], gsems[m % 2])

        def sct(m):
            return pltpu.async_copy(
                rbufs[m % 2], acc.at[sib.at[m]], ssems[m % 2], add=True)

        def body(j, nreal):
            pltpu.sync_copy(edg_hbm.at[1 - c, s, pl.ds(8 * j, 8)], sib)
            g = {0: gat(j, 0)}
            if nreal > 1:
                g[1] = gat(j, 1)
            for m in range(nreal):
                g[m].wait()
                sm = sct(m)
                sm.wait()
                if m + 2 <= nreal - 1:
                    g[m + 2] = gat(j, m + 2)

        def chunk8(j, carry):
            body(j, 8)
            return carry

        lax.fori_loop(0, 15, chunk8, 0)
        body(15, 5)
        plsc.subcore_barrier()

        def wch(j, carry):
            pltpu.async_copy(acc.at[pl.ds(rbase + 80 * j, 80)],
                             out_hbm.at[c, p, pl.ds(rbase + 80 * j, 80)],
                             wsem)
            return carry

        def wdrain2(j, carry):
            pltpu.make_async_copy(z_hbm, acc.at[pl.ds(rbase, 80)],
                                  wsem).wait()
            return carry

        lax.fori_loop(0, nch, wch, 0)
        lax.fori_loop(0, nch, wdrain2, 0)
        plsc.subcore_barrier()


def _tc_dense_body(x_ref, wc_ref, deg_ref, hd_ref, t_ref, dis_ref,
                   dsum_ref):
    i = pl.program_id(0)
    h = jnp.dot(x_ref[...], wc_ref[...], preferred_element_type=jnp.float32)
    degsl = jnp.sum(hd_ref[...], axis=1) + 1.0
    dis = lax.rsqrt(degsl)
    hs = h * dis[:, None]
    for p in range(2):
        t_ref[0, p] = hs[:, 128 * p:128 * (p + 1)]
        t_ref[1, p] = h[:, 128 * p:128 * (p + 1)]
    dis_ref[...] = dis[:, None]
    d_i = deg_ref[...]

    @pl.when(i == 0)
    def _():
        dsum_ref[...] = jnp.zeros_like(dsum_ref)

    dsum_ref[...] += jnp.sum(d_i.astype(jnp.float32))


def _tc_dense(x, W_conv, degree, hd):
    return pl.pallas_call(
        _tc_dense_body,
        grid=(GRID,),
        in_specs=[
            pl.BlockSpec((BLK, C), lambda i: (i, 0)),
            pl.BlockSpec((C, C), lambda i: (0, 0)),
            pl.BlockSpec((BLK, 1), lambda i: (i, 0)),
            pl.BlockSpec((BLK, 32), lambda i: (i, 0)),
        ],
        out_specs=[
            pl.BlockSpec((2, 2, BLK, 128), lambda i: (0, 0, i, 0)),
            pl.BlockSpec((BLK, 1), lambda i: (i, 0)),
            pl.BlockSpec((1, 1), lambda i: (0, 0)),
        ],
        out_shape=[
            jax.ShapeDtypeStruct((2, 2, N, 128), jnp.float32),
            jax.ShapeDtypeStruct((N, 1), jnp.float32),
            jax.ShapeDtypeStruct((1, 1), jnp.float32),
        ],
    )(x, W_conv, degree, hd)


def _tc_final_body(ac_ref, t_ref, dis_ref, deg_ref, pe_ref, wg_ref,
                   wb_ref, bg_ref, bb_ref, wa_ref, wr_ref, hi_ref, dsum_ref,
                   out_ref, lb_ref, lf_ref):
    i = pl.program_id(0)
    agg = jnp.concatenate([ac_ref[1, 0], ac_ref[1, 1]], axis=1) * (DM ** 0.5)
    deg_f = deg_ref[...].astype(jnp.float32)
    safe = jnp.where(deg_f == 0, 1.0, deg_f)
    i_n = jnp.where(deg_f == 0, 0.0, agg / safe)
    A = jnp.dot(i_n, wa_ref[...], preferred_element_type=jnp.float32)
    Rv = jnp.dot(i_n, wr_ref[...], preferred_element_type=jnp.float32)
    iota = lax.broadcasted_iota(jnp.int32, (BLK, 128), 1)
    onehot = (deg_ref[...] == iota).astype(jnp.float32)
    m_dv = jnp.dot(onehot, pe_ref[...], preferred_element_type=jnp.float32)

    def lrelu(v):
        return jnp.where(v >= 0, v, 0.01 * v)

    gam = lrelu(
        jnp.dot(m_dv, wg_ref[...], preferred_element_type=jnp.float32)
        + bg_ref[...])
    bet = lrelu(
        jnp.dot(m_dv, wb_ref[...], preferred_element_type=jnp.float32)
        + bb_ref[...])
    gp1 = gam + 1.0
    b_add = gp1 * A + bet
    b_rev = gp1 * Rv + bet
    Kv = dsum_ref[0, 0] * (K_HYP / N)
    R = (deg_f < Kv).astype(jnp.float32)
    bias = OMEGA * (R * b_add - (1.0 - R) * b_rev)
    hfull = jnp.concatenate([t_ref[0, 0], t_ref[0, 1]], axis=1)
    conv1 = jnp.concatenate([ac_ref[0, 0], ac_ref[0, 1]], axis=1)
    dis = dis_ref[...]
    out_ref[...] = conv1 * dis + hfull * (dis * dis) + bias
    na = jnp.sqrt(jnp.sum(b_add * b_add, axis=1, keepdims=True)) * R
    nr = jnp.sqrt(jnp.sum(b_rev * b_rev, axis=1, keepdims=True)) * (1.0 - R)
    ng = jnp.sqrt(jnp.sum(gam * gam, axis=1))
    nbv = jnp.sqrt(jnp.sum(bet * bet, axis=1))
    cnt = jnp.sum(hi_ref[...], axis=1)

    @pl.when(i == 0)
    def _():
        lb_ref[...] = jnp.zeros_like(lb_ref)
        lf_ref[...] = jnp.zeros_like(lf_ref)

    lb_ref[...] += jnp.sum(cnt * (na + nr)[:, 0])
    lf_ref[...] += jnp.sum(cnt * (ng + nbv))


def _tc_final(ac, t, dis, degree, pe, W_gamma, W_beta, b_gamma, b_beta,
              W_add, W_rev, hi, dsum):
    return pl.pallas_call(
        _tc_final_body,
        grid=(GRID,),
        in_specs=[
            pl.BlockSpec((2, 2, BLK, 128), lambda i: (0, 0, i, 0)),
            pl.BlockSpec((1, 2, BLK, 128), lambda i: (1, 0, i, 0)),
            pl.BlockSpec((BLK, 1), lambda i: (i, 0)),
            pl.BlockSpec((BLK, 1), lambda i: (i, 0)),
            pl.BlockSpec((128, DM), lambda i: (0, 0)),
            pl.BlockSpec((DM, C), lambda i: (0, 0)),
            pl.BlockSpec((DM, C), lambda i: (0, 0)),
            pl.BlockSpec((1, C), lambda i: (0, 0)),
            pl.BlockSpec((1, C), lambda i: (0, 0)),
            pl.BlockSpec((C, C), lambda i: (0, 0)),
            pl.BlockSpec((C, C), lambda i: (0, 0)),
            pl.BlockSpec((BLK, 32), lambda i: (i, 0)),
            pl.BlockSpec((1, 1), lambda i: (0, 0)),
        ],
        out_specs=[
            pl.BlockSpec((BLK, C), lambda i: (i, 0)),
            pl.BlockSpec((1, 1), lambda i: (0, 0)),
            pl.BlockSpec((1, 1), lambda i: (0, 0)),
        ],
        out_shape=[
            jax.ShapeDtypeStruct((N, C), jnp.float32),
            jax.ShapeDtypeStruct((1, 1), jnp.float32),
            jax.ShapeDtypeStruct((1, 1), jnp.float32),
        ],
    )(ac, t, dis, degree, pe, W_gamma, W_beta, b_gamma, b_beta,
      W_add, W_rev, hi, dsum)


def kernel(x, adj, degree, idx, edge, W_conv, W_gamma, W_beta, b_gamma,
           b_beta, W_add, W_rev, PE):
    src = adj[0]
    dst = adj[1]
    pad = N + (jnp.arange(3840, dtype=jnp.int32) % 16)
    dstp = jnp.concatenate([dst, pad]).reshape(2, 16, 1, 5120)
    idxp = idx.reshape(2, 16, 1, 64)
    epad = jnp.zeros((2, 16, 3, 80), jnp.int32)
    edg = jnp.concatenate([adj.reshape(2, 16, 125, 80), epad], axis=2)

    histd, histi = _sc_hist(dstp, idxp)
    hd = histd.reshape(32, HN)[:, :N].T
    hi = histi.reshape(32, HN)[:, :N].T

    T, dis, dsum = _tc_dense(x, W_conv, degree, hd)

    z = jnp.zeros((80, 128), jnp.float32)
    ac = _sc_scatter(T, edg, z)

    output, lb, lf = _tc_final(ac, T, dis, degree, PE[:128], W_gamma,
                               W_beta, b_gamma, b_beta, W_add, W_rev, hi,
                               dsum)
    nb = float(B_IDX)
    return output, lb[0, 0] / nb, lf[0, 0] / nb


# speedup vs baseline: 1.1660x; 1.0147x over previous
import functools

import jax
import jax.numpy as jnp
from jax import lax
from jax.experimental import pallas as pl
from jax.experimental.pallas import tpu as pltpu
from jax.experimental.pallas import tpu_sc as plsc

N = 10000
E = 160000
C = 256
DM = 64
OMEGA = 0.1
K_HYP = 1.0
B_IDX = 2048
HN = 10240
BLK = 1000
GRID = N // BLK

_mesh = plsc.VectorSubcoreMesh(core_axis_name="c", subcore_axis_name="s")


@functools.partial(
    pl.kernel,
    out_type=[jax.ShapeDtypeStruct((2, 16, 1, HN), jnp.float32),
              jax.ShapeDtypeStruct((2, 16, 1, HN), jnp.float32)],
    mesh=_mesh,
    scratch_types=[
        pltpu.VMEM((1, 5120), jnp.int32),
        pltpu.VMEM((1, 64), jnp.int32),
        pltpu.VMEM((HN,), jnp.float32),
        pltpu.VMEM((HN,), jnp.float32),
    ],
    compiler_params=pltpu.CompilerParams(needs_layout_passes=False),
)
def _sc_hist(dstp_hbm, idxp_hbm, outd_hbm, outi_hbm, dst_v, idx_v,
             hd_v, hi_v):
    c = lax.axis_index("c")
    s = lax.axis_index("s")

    def zero(j, carry):
        hd_v[pl.ds(16 * j, 16)] = jnp.zeros((16,), jnp.float32)
        hi_v[pl.ds(16 * j, 16)] = jnp.zeros((16,), jnp.float32)
        return carry

    lax.fori_loop(0, HN // 16, zero, 0)
    pltpu.sync_copy(dstp_hbm.at[c, s], dst_v)
    pltpu.sync_copy(idxp_hbm.at[c, s], idx_v)

    ones = jnp.full((16,), 1.0, jnp.float32)

    def accum_d(j, carry):
        v = dst_v[0, pl.ds(16 * j, 16)]
        plsc.addupdate_scatter(hd_v, [v], ones)
        return carry

    lax.fori_loop(0, 5120 // 16, accum_d, 0)

    def accum_i(j, carry):
        v = idx_v[0, pl.ds(16 * j, 16)]
        plsc.addupdate_scatter(hi_v, [v], ones)
        return carry

    lax.fori_loop(0, 64 // 16, accum_i, 0)
    pltpu.sync_copy(hd_v, outd_hbm.at[c, s, 0])
    pltpu.sync_copy(hi_v, outi_hbm.at[c, s, 0])


@functools.partial(
    pl.kernel,
    out_type=jax.ShapeDtypeStruct((2, 2, N, 128), jnp.float32),
    mesh=_mesh,
    scratch_types=[
        pltpu.VMEM((128, 80), jnp.int32),
        pltpu.VMEM((8, 80), jnp.int32),
        pltpu.VMEM((80, 128), jnp.float32),
        pltpu.VMEM((80, 128), jnp.float32),
        pltpu.VMEM_SHARED((N, 128), jnp.float32),
        pltpu.SemaphoreType.DMA,
        pltpu.SemaphoreType.DMA,
        pltpu.SemaphoreType.DMA,
        pltpu.SemaphoreType.DMA,
        pltpu.SemaphoreType.DMA,
        pltpu.SemaphoreType.DMA,
        pltpu.SemaphoreType.DMA,
    ],
)
def _sc_scatter(t_hbm, edg_hbm, z_hbm, out_hbm,
                gi_v, sib, rbA, rbB, acc, g0, g1, g2, g3, s0, s1, wsem):
    rbufs = [rbA, rbB]
    gsems = [[g0, g1], [g2, g3]]
    ssems = [s0, s1]
    QS = [(0, 40), (40, 40)]
    c = lax.axis_index("c")
    s = lax.axis_index("s")
    pltpu.sync_copy(edg_hbm.at[c, s], gi_v)
    rbase = s * 640
    nch = jnp.where(s < 15, 8, 5)
    for p in range(2):

        def zch(j, carry):
            pltpu.async_copy(z_hbm, acc.at[pl.ds(rbase + 80 * j, 80)], wsem)
            return carry

        def wdrain(j, carry):
            pltpu.make_async_copy(z_hbm, acc.at[pl.ds(rbase, 80)],
                                  wsem).wait()
            return carry

        lax.fori_loop(0, nch, zch, 0)
        lax.fori_loop(0, nch, wdrain, 0)
        plsc.subcore_barrier()

        def gat(j, m):
            ga = pltpu.async_copy(
                t_hbm.at[c, p].at[gi_v.at[8 * j + m, pl.ds(0, 40)]],
                rbufs[m % 2].at[pl.ds(0, 40)], gsems[m % 2][0])
            gb = pltpu.async_copy(
                t_hbm.at[c, p].at[gi_v.at[8 * j + m, pl.ds(40, 40)]],
                rbufs[m % 2].at[pl.ds(40, 40)], gsems[m % 2][1])
            return ga, gb

        def sct(m):
            return pltpu.async_copy(
                rbufs[m % 2], acc.at[sib.at[m]], ssems[m % 2], add=True)

        def body(j, nreal):
            pltpu.sync_copy(edg_hbm.at[1 - c, s, pl.ds(8 * j, 8)], sib)
            g = {0: gat(j, 0)}
            if nreal > 1:
                g[1] = gat(j, 1)
            for m in range(nreal):
                g[m][0].wait()
                g[m][1].wait()
                sm = sct(m)
                sm.wait()
                if m + 2 <= nreal - 1:
                    g[m + 2] = gat(j, m + 2)

        def chunk8(j, carry):
            body(j, 8)
            return carry

        lax.fori_loop(0, 15, chunk8, 0)
        body(15, 5)
        plsc.subcore_barrier()

        def wch(j, carry):
            pltpu.async_copy(acc.at[pl.ds(rbase + 80 * j, 80)],
                             out_hbm.at[c, p, pl.ds(rbase + 80 * j, 80)],
                             wsem)
            return carry

        def wdrain2(j, carry):
            pltpu.make_async_copy(z_hbm, acc.at[pl.ds(rbase, 80)],
                                  wsem).wait()
            return carry

        lax.fori_loop(0, nch, wch, 0)
        lax.fori_loop(0, nch, wdrain2, 0)
        plsc.subcore_barrier()


def _tc_dense_body(x_ref, wc_ref, deg_ref, hd_ref, t_ref, dis_ref,
                   dsum_ref):
    i = pl.program_id(0)
    h = jnp.dot(x_ref[...], wc_ref[...], preferred_element_type=jnp.float32)
    degsl = jnp.sum(hd_ref[...], axis=1) + 1.0
    dis = lax.rsqrt(degsl)
    hs = h * dis[:, None]
    for p in range(2):
        t_ref[0, p] = hs[:, 128 * p:128 * (p + 1)]
        t_ref[1, p] = h[:, 128 * p:128 * (p + 1)]
    dis_ref[...] = dis[:, None]
    d_i = deg_ref[...]

    @pl.when(i == 0)
    def _():
        dsum_ref[...] = jnp.zeros_like(dsum_ref)

    dsum_ref[...] += jnp.sum(d_i.astype(jnp.float32))


def _tc_dense(x, W_conv, degree, hd):
    return pl.pallas_call(
        _tc_dense_body,
        grid=(GRID,),
        in_specs=[
            pl.BlockSpec((BLK, C), lambda i: (i, 0)),
            pl.BlockSpec((C, C), lambda i: (0, 0)),
            pl.BlockSpec((BLK, 1), lambda i: (i, 0)),
            pl.BlockSpec((BLK, 32), lambda i: (i, 0)),
        ],
        out_specs=[
            pl.BlockSpec((2, 2, BLK, 128), lambda i: (0, 0, i, 0)),
            pl.BlockSpec((BLK, 1), lambda i: (i, 0)),
            pl.BlockSpec((1, 1), lambda i: (0, 0)),
        ],
        out_shape=[
            jax.ShapeDtypeStruct((2, 2, N, 128), jnp.float32),
            jax.ShapeDtypeStruct((N, 1), jnp.float32),
            jax.ShapeDtypeStruct((1, 1), jnp.float32),
        ],
    )(x, W_conv, degree, hd)


def _tc_final_body(ac_ref, t_ref, dis_ref, deg_ref, pe_ref, wg_ref,
                   wb_ref, bg_ref, bb_ref, wa_ref, wr_ref, hi_ref, dsum_ref,
                   out_ref, lb_ref, lf_ref):
    i = pl.program_id(0)
    agg = jnp.concatenate([ac_ref[1, 0], ac_ref[1, 1]], axis=1) * (DM ** 0.5)
    deg_f = deg_ref[...].astype(jnp.float32)
    safe = jnp.where(deg_f == 0, 1.0, deg_f)
    i_n = jnp.where(deg_f == 0, 0.0, agg / safe)
    A = jnp.dot(i_n, wa_ref[...], preferred_element_type=jnp.float32)
    Rv = jnp.dot(i_n, wr_ref[...], preferred_element_type=jnp.float32)
    iota = lax.broadcasted_iota(jnp.int32, (BLK, 128), 1)
    onehot = (deg_ref[...] == iota).astype(jnp.float32)
    m_dv = jnp.dot(onehot, pe_ref[...], preferred_element_type=jnp.float32)

    def lrelu(v):
        return jnp.where(v >= 0, v, 0.01 * v)

    gam = lrelu(
        jnp.dot(m_dv, wg_ref[...], preferred_element_type=jnp.float32)
        + bg_ref[...])
    bet = lrelu(
        jnp.dot(m_dv, wb_ref[...], preferred_element_type=jnp.float32)
        + bb_ref[...])
    gp1 = gam + 1.0
    b_add = gp1 * A + bet
    b_rev = gp1 * Rv + bet
    Kv = dsum_ref[0, 0] * (K_HYP / N)
    R = (deg_f < Kv).astype(jnp.float32)
    bias = OMEGA * (R * b_add - (1.0 - R) * b_rev)
    hfull = jnp.concatenate([t_ref[0, 0], t_ref[0, 1]], axis=1)
    conv1 = jnp.concatenate([ac_ref[0, 0], ac_ref[0, 1]], axis=1)
    dis = dis_ref[...]
    out_ref[...] = conv1 * dis + hfull * (dis * dis) + bias
    na = jnp.sqrt(jnp.sum(b_add * b_add, axis=1, keepdims=True)) * R
    nr = jnp.sqrt(jnp.sum(b_rev * b_rev, axis=1, keepdims=True)) * (1.0 - R)
    ng = jnp.sqrt(jnp.sum(gam * gam, axis=1))
    nbv = jnp.sqrt(jnp.sum(bet * bet, axis=1))
    cnt = jnp.sum(hi_ref[...], axis=1)

    @pl.when(i == 0)
    def _():
        lb_ref[...] = jnp.zeros_like(lb_ref)
        lf_ref[...] = jnp.zeros_like(lf_ref)

    lb_ref[...] += jnp.sum(cnt * (na + nr)[:, 0])
    lf_ref[...] += jnp.sum(cnt * (ng + nbv))


def _tc_final(ac, t, dis, degree, pe, W_gamma, W_beta, b_gamma, b_beta,
              W_add, W_rev, hi, dsum):
    return pl.pallas_call(
        _tc_final_body,
        grid=(GRID,),
        in_specs=[
            pl.BlockSpec((2, 2, BLK, 128), lambda i: (0, 0, i, 0)),
            pl.BlockSpec((1, 2, BLK, 128), lambda i: (1, 0, i, 0)),
            pl.BlockSpec((BLK, 1), lambda i: (i, 0)),
            pl.BlockSpec((BLK, 1), lambda i: (i, 0)),
            pl.BlockSpec((128, DM), lambda i: (0, 0)),
            pl.BlockSpec((DM, C), lambda i: (0, 0)),
            pl.BlockSpec((DM, C), lambda i: (0, 0)),
            pl.BlockSpec((1, C), lambda i: (0, 0)),
            pl.BlockSpec((1, C), lambda i: (0, 0)),
            pl.BlockSpec((C, C), lambda i: (0, 0)),
            pl.BlockSpec((C, C), lambda i: (0, 0)),
            pl.BlockSpec((BLK, 32), lambda i: (i, 0)),
            pl.BlockSpec((1, 1), lambda i: (0, 0)),
        ],
        out_specs=[
            pl.BlockSpec((BLK, C), lambda i: (i, 0)),
            pl.BlockSpec((1, 1), lambda i: (0, 0)),
            pl.BlockSpec((1, 1), lambda i: (0, 0)),
        ],
        out_shape=[
            jax.ShapeDtypeStruct((N, C), jnp.float32),
            jax.ShapeDtypeStruct((1, 1), jnp.float32),
            jax.ShapeDtypeStruct((1, 1), jnp.float32),
        ],
    )(ac, t, dis, degree, pe, W_gamma, W_beta, b_gamma, b_beta,
      W_add, W_rev, hi, dsum)


def kernel(x, adj, degree, idx, edge, W_conv, W_gamma, W_beta, b_gamma,
           b_beta, W_add, W_rev, PE):
    src = adj[0]
    dst = adj[1]
    pad = N + (jnp.arange(3840, dtype=jnp.int32) % 16)
    dstp = jnp.concatenate([dst, pad]).reshape(2, 16, 1, 5120)
    idxp = idx.reshape(2, 16, 1, 64)
    epad = jnp.zeros((2, 16, 3, 80), jnp.int32)
    edg = jnp.concatenate([adj.reshape(2, 16, 125, 80), epad], axis=2)

    histd, histi = _sc_hist(dstp, idxp)
    hd = histd.reshape(32, HN)[:, :N].T
    hi = histi.reshape(32, HN)[:, :N].T

    T, dis, dsum = _tc_dense(x, W_conv, degree, hd)

    z = jnp.zeros((80, 128), jnp.float32)
    ac = _sc_scatter(T, edg, z)

    output, lb, lf = _tc_final(ac, T, dis, degree, PE[:128], W_gamma,
                               W_beta, b_gamma, b_beta, W_add, W_rev, hi,
                               dsum)
    nb = float(B_IDX)
    return output, lb[0, 0] / nb, lf[0, 0] / nb
